# baseline (speedup 1.0000x reference)
import functools
import jax
import jax.numpy as jnp
from jax import lax
from jax.experimental import pallas as pl
from jax.experimental.pallas import tpu as pltpu
from jax.experimental.pallas import tpu_sc as plsc

_CHUNKS = 4


def _make_sc_copy(S, H, dtype):
    info = plsc.get_sparse_core_info()
    NC, NS = info.num_cores, info.num_subcores
    NW = NC * NS
    rows_per_w = S // NW
    rows_per_c = rows_per_w // _CHUNKS
    mesh = plsc.VectorSubcoreMesh(core_axis_name="c", subcore_axis_name="s")

    @functools.partial(
        pl.kernel,
        out_type=jax.ShapeDtypeStruct((1, S, H), dtype),
        mesh=mesh,
        scratch_types=[
            pltpu.VMEM((rows_per_w, H), dtype),
            pltpu.SemaphoreType.DMA((_CHUNKS,)),
            pltpu.SemaphoreType.DMA((_CHUNKS,)),
        ],
    )
    def sc_copy(src_hbm, out_hbm, buf_v, g_sems, s_sems):
        wid = lax.axis_index("s") * NC + lax.axis_index("c")
        base = wid * rows_per_w
        gathers = []
        for i in range(_CHUNKS):
            gathers.append(
                pltpu.async_copy(
                    src_hbm.at[0, pl.ds(base + i * rows_per_c, rows_per_c), :],
                    buf_v.at[pl.ds(i * rows_per_c, rows_per_c), :],
                    g_sems.at[i],
                )
            )
        scatters = []
        for i in range(_CHUNKS):
            gathers[i].wait()
            scatters.append(
                pltpu.async_copy(
                    buf_v.at[pl.ds(i * rows_per_c, rows_per_c), :],
                    out_hbm.at[0, pl.ds(base + i * rows_per_c, rows_per_c), :],
                    s_sems.at[i],
                )
            )
        for i in range(_CHUNKS):
            scatters[i].wait()

    return sc_copy


def kernel(last_hidden_state, input_ids):
    del input_ids
    B, S, H = last_hidden_state.shape
    return _make_sc_copy(S, H, last_hidden_state.dtype)(last_hidden_state)
